# baseline (speedup 1.0000x reference)
import jax
import jax.numpy as jnp
from jax.experimental import pallas as pl
from jax.experimental.pallas import tpu as pltpu

_KS = 30
_STR = 5
_LU = 6000
_P1 = 1195
_P1_PAD = 1280
_FEAT = 234
_FEAT_PAD = 256
_K1 = 672
_NBLK1 = 10


def _band_matrix(w, n_rows, n_cols, col_limit):
    r = jnp.arange(n_rows)[:, None]
    c = jnp.arange(n_cols)[None, :]
    tap = r - _STR * c
    mask = (tap >= 0) & (tap < _KS) & (c < col_limit)
    return jnp.where(mask, w[jnp.clip(tap, 0, _KS - 1)], 0.0).astype(jnp.float32)


def _net_kernel(b1_ref, b2_ref, x_ref, B1_ref, B2_ref, woutT_ref, bout_ref,
                out_ref, p1):
    b1 = b1_ref[0]
    b2 = b2_ref[0]

    for j in range(_NBLK1):
        lo = 640 * j
        span = min(_K1, _LU - lo)
        acc = jnp.dot(x_ref[:, 0, lo:lo + span], B1_ref[0:span, :],
                      preferred_element_type=jnp.float32)
        p1[:, 128 * j:128 * (j + 1)] = jnp.maximum(acc + b1, 0.0)

    acc2 = jnp.dot(p1[...], B2_ref[...], preferred_element_type=jnp.float32)
    h = jnp.maximum(acc2 + b2, 0.0)

    out_ref[...] = (jnp.dot(h, woutT_ref[...],
                            preferred_element_type=jnp.float32)
                    + bout_ref[...])


def _softmax_dim0_kernel(z_ref, o_ref):
    z = z_ref[...]
    m = jnp.max(z, axis=0, keepdims=True)
    e = jnp.exp(z - m)
    o_ref[...] = e / jnp.sum(e, axis=0, keepdims=True)


def kernel(x, w1, b1, w2, b2, wout, bout):
    n, ch, length = x.shape
    assert ch == 1 and length >= _LU
    x3 = x.astype(jnp.float32)
    if length > _LU:
        x3 = x3[:, :, :_LU]

    tb = n if n <= 64 else 64
    assert tb % 8 == 0
    n_pad = ((n + tb - 1) // tb) * tb
    if n_pad != n:
        x3 = jnp.pad(x3, ((0, n_pad - n), (0, 0), (0, 0)))

    B1 = _band_matrix(w1.astype(jnp.float32), _K1, 128, 128)
    B2 = _band_matrix(w2.astype(jnp.float32), _P1_PAD, _FEAT_PAD, _FEAT)
    woutT = jnp.zeros((_FEAT_PAD, 5), jnp.float32).at[:_FEAT].set(
        wout.astype(jnp.float32).T)

    logits_pad = pl.pallas_call(
        _net_kernel,
        out_shape=jax.ShapeDtypeStruct((n_pad, 5), jnp.float32),
        grid=(n_pad // tb,),
        in_specs=[
            pl.BlockSpec(memory_space=pltpu.MemorySpace.SMEM),
            pl.BlockSpec(memory_space=pltpu.MemorySpace.SMEM),
            pl.BlockSpec((tb, 1, _LU), lambda i: (i, 0, 0)),
            pl.BlockSpec(memory_space=pltpu.MemorySpace.VMEM),
            pl.BlockSpec(memory_space=pltpu.MemorySpace.VMEM),
            pl.BlockSpec(memory_space=pltpu.MemorySpace.VMEM),
            pl.BlockSpec(memory_space=pltpu.MemorySpace.VMEM),
        ],
        out_specs=pl.BlockSpec((tb, 5), lambda i: (i, 0)),
        scratch_shapes=[pltpu.VMEM((tb, _P1_PAD), jnp.float32)],
        compiler_params=pltpu.CompilerParams(
            dimension_semantics=("parallel",),
            vmem_limit_bytes=56 * 1024 * 1024),
    )(b1.astype(jnp.float32), b2.astype(jnp.float32),
      x3, B1, B2, woutT, bout.reshape(1, 5).astype(jnp.float32))

    logits = logits_pad[:n]

    probs = pl.pallas_call(
        _softmax_dim0_kernel,
        out_shape=jax.ShapeDtypeStruct((n, 5), jnp.float32),
        in_specs=[pl.BlockSpec(memory_space=pltpu.MemorySpace.VMEM)],
        out_specs=pl.BlockSpec(memory_space=pltpu.MemorySpace.VMEM),
    )(logits)

    return probs.reshape(n, 1, 5)


# speedup vs baseline: 1.3001x; 1.3001x over previous
import jax
import jax.numpy as jnp
from jax.experimental import pallas as pl
from jax.experimental.pallas import tpu as pltpu

_KS = 30
_STR = 5
_LU = 6000
_P1_PAD = 1280
_FEAT = 234
_FEAT_PAD = 256
_K1 = 672
_NBLK1 = 10
_CHUNK = 512


def _band_matrix(w, n_rows, n_cols, col_limit):
    r = jnp.arange(n_rows)[:, None]
    c = jnp.arange(n_cols)[None, :]
    tap = r - _STR * c
    mask = (tap >= 0) & (tap < _KS) & (c < col_limit)
    return jnp.where(mask, w[jnp.clip(tap, 0, _KS - 1)], 0.0).astype(jnp.float32)


def _net_kernel(b1_ref, b2_ref, x_ref, B1_ref, B2_ref, woutT_ref, bout_ref,
                out_ref, p1):
    b1 = b1_ref[0]
    b2 = b2_ref[0]

    for j in range(_NBLK1):
        lo = 640 * j
        span = min(_K1, _LU - lo)
        acc = jnp.dot(x_ref[:, lo:lo + span], B1_ref[0:span, :],
                      preferred_element_type=jnp.float32)
        p1[:, 128 * j:128 * (j + 1)] = jnp.maximum(acc + b1, 0.0)

    acc2 = jnp.dot(p1[...], B2_ref[...], preferred_element_type=jnp.float32)
    h = jnp.maximum(acc2 + b2, 0.0)

    out_ref[...] = (jnp.dot(h, woutT_ref[...],
                            preferred_element_type=jnp.float32)
                    + bout_ref[...])


def _softmax_dim0_kernel(z_ref, o_ref):
    z = z_ref[...]
    m = jnp.max(z, axis=0, keepdims=True)
    e = jnp.exp(z - m)
    o_ref[...] = e / jnp.sum(e, axis=0, keepdims=True)


def kernel(x, w1, b1, w2, b2, wout, bout):
    n, ch, length = x.shape
    assert ch == 1 and length >= _LU

    B1 = _band_matrix(w1.astype(jnp.float32), _K1, 128, 128)
    B2 = _band_matrix(w2.astype(jnp.float32), _P1_PAD, _FEAT_PAD, _FEAT)
    woutT = jnp.zeros((_FEAT_PAD, 5), jnp.float32).at[:_FEAT].set(
        wout.astype(jnp.float32).T)
    b1f = b1.astype(jnp.float32)
    b2f = b2.astype(jnp.float32)
    boutf = bout.reshape(1, 5).astype(jnp.float32)

    def run_chunk(xc):
        nc = xc.shape[0]
        tb = nc if nc <= 128 else 128
        assert tb % 8 == 0
        nc_pad = ((nc + tb - 1) // tb) * tb
        x2 = xc.reshape(nc, length)[:, :_LU].astype(jnp.float32)
        if nc_pad != nc:
            x2 = jnp.pad(x2, ((0, nc_pad - nc), (0, 0)))
        logits_pad = pl.pallas_call(
            _net_kernel,
            out_shape=jax.ShapeDtypeStruct((nc_pad, 5), jnp.float32),
            grid=(nc_pad // tb,),
            in_specs=[
                pl.BlockSpec(memory_space=pltpu.MemorySpace.SMEM),
                pl.BlockSpec(memory_space=pltpu.MemorySpace.SMEM),
                pl.BlockSpec((tb, _LU), lambda i: (i, 0)),
                pl.BlockSpec(memory_space=pltpu.MemorySpace.VMEM),
                pl.BlockSpec(memory_space=pltpu.MemorySpace.VMEM),
                pl.BlockSpec(memory_space=pltpu.MemorySpace.VMEM),
                pl.BlockSpec(memory_space=pltpu.MemorySpace.VMEM),
            ],
            out_specs=pl.BlockSpec((tb, 5), lambda i: (i, 0)),
            scratch_shapes=[pltpu.VMEM((tb, _P1_PAD), jnp.float32)],
            compiler_params=pltpu.CompilerParams(
                dimension_semantics=("parallel",),
                vmem_limit_bytes=48 * 1024 * 1024),
        )(b1f, b2f, x2, B1, B2, woutT, boutf)
        return logits_pad[:nc]

    if n % _CHUNK == 0 and n > _CHUNK:
        parts = [run_chunk(x[c:c + _CHUNK]) for c in range(0, n, _CHUNK)]
        logits = jnp.concatenate(parts, axis=0)
    else:
        logits = run_chunk(x)

    probs = pl.pallas_call(
        _softmax_dim0_kernel,
        out_shape=jax.ShapeDtypeStruct((n, 5), jnp.float32),
        in_specs=[pl.BlockSpec(memory_space=pltpu.MemorySpace.VMEM)],
        out_specs=pl.BlockSpec(memory_space=pltpu.MemorySpace.VMEM),
    )(logits)

    return probs.reshape(n, 1, 5)


# speedup vs baseline: 2.4048x; 1.8496x over previous
import jax
import jax.numpy as jnp
from jax.experimental import pallas as pl
from jax.experimental.pallas import tpu as pltpu

_KS = 30
_STR = 5
_LU = 6000
_P1_PAD = 1280
_FEAT = 234
_FEAT_PAD = 256
_K1 = 672
_NBLK1 = 10
_CHUNK = 512


def _weights_kernel(w1_ref, w2_ref, B1_ref, B2_ref):
    tap1 = (jax.lax.broadcasted_iota(jnp.int32, B1_ref.shape, 0)
            - _STR * jax.lax.broadcasted_iota(jnp.int32, B1_ref.shape, 1))
    tap2 = (jax.lax.broadcasted_iota(jnp.int32, B2_ref.shape, 0)
            - _STR * jax.lax.broadcasted_iota(jnp.int32, B2_ref.shape, 1))
    col2 = jax.lax.broadcasted_iota(jnp.int32, B2_ref.shape, 1)
    acc1 = jnp.zeros(B1_ref.shape, jnp.float32)
    acc2 = jnp.zeros(B2_ref.shape, jnp.float32)
    for t in range(_KS):
        acc1 = acc1 + jnp.where(tap1 == t, w1_ref[t], 0.0)
        acc2 = acc2 + jnp.where((tap2 == t) & (col2 < _FEAT), w2_ref[t], 0.0)
    B1_ref[...] = acc1
    B2_ref[...] = acc2


def _net_kernel(b1_ref, b2_ref, x_ref, B1_ref, B2_ref, woutT_ref, bout_ref,
                out_ref, p1):
    b1 = b1_ref[0]
    b2 = b2_ref[0]

    for j in range(_NBLK1):
        lo = 640 * j
        span = min(_K1, _LU - lo)
        acc = jnp.dot(x_ref[:, lo:lo + span], B1_ref[0:span, :],
                      preferred_element_type=jnp.float32)
        p1[:, 128 * j:128 * (j + 1)] = jnp.maximum(acc + b1, 0.0)

    acc2 = jnp.dot(p1[...], B2_ref[...], preferred_element_type=jnp.float32)
    h = jnp.maximum(acc2 + b2, 0.0)

    out_ref[...] = (jnp.dot(h, woutT_ref[...],
                            preferred_element_type=jnp.float32)
                    + bout_ref[...])


def _softmax_dim0_kernel(z_ref, o_ref):
    z = z_ref[...]
    m = jnp.max(z, axis=0, keepdims=True)
    e = jnp.exp(z - m)
    o_ref[...] = e / jnp.sum(e, axis=0, keepdims=True)


def kernel(x, w1, b1, w2, b2, wout, bout):
    n, ch, length = x.shape
    assert ch == 1 and length >= _LU

    B1, B2 = pl.pallas_call(
        _weights_kernel,
        out_shape=(jax.ShapeDtypeStruct((_K1, 128), jnp.float32),
                   jax.ShapeDtypeStruct((_P1_PAD, _FEAT_PAD), jnp.float32)),
        in_specs=[pl.BlockSpec(memory_space=pltpu.MemorySpace.SMEM),
                  pl.BlockSpec(memory_space=pltpu.MemorySpace.SMEM)],
        out_specs=(pl.BlockSpec(memory_space=pltpu.MemorySpace.VMEM),
                   pl.BlockSpec(memory_space=pltpu.MemorySpace.VMEM)),
    )(w1.astype(jnp.float32), w2.astype(jnp.float32))
    woutT = jnp.zeros((_FEAT_PAD, 5), jnp.float32).at[:_FEAT].set(
        wout.astype(jnp.float32).T)
    b1f = b1.astype(jnp.float32)
    b2f = b2.astype(jnp.float32)
    boutf = bout.reshape(1, 5).astype(jnp.float32)

    def run_chunk(xc):
        nc = xc.shape[0]
        tb = nc if nc <= 128 else 128
        assert tb % 8 == 0
        nc_pad = ((nc + tb - 1) // tb) * tb
        x2 = xc.reshape(nc, length)[:, :_LU].astype(jnp.float32)
        if nc_pad != nc:
            x2 = jnp.pad(x2, ((0, nc_pad - nc), (0, 0)))
        logits_pad = pl.pallas_call(
            _net_kernel,
            out_shape=jax.ShapeDtypeStruct((nc_pad, 5), jnp.float32),
            grid=(nc_pad // tb,),
            in_specs=[
                pl.BlockSpec(memory_space=pltpu.MemorySpace.SMEM),
                pl.BlockSpec(memory_space=pltpu.MemorySpace.SMEM),
                pl.BlockSpec((tb, _LU), lambda i: (i, 0)),
                pl.BlockSpec(memory_space=pltpu.MemorySpace.VMEM),
                pl.BlockSpec(memory_space=pltpu.MemorySpace.VMEM),
                pl.BlockSpec(memory_space=pltpu.MemorySpace.VMEM),
                pl.BlockSpec(memory_space=pltpu.MemorySpace.VMEM),
            ],
            out_specs=pl.BlockSpec((tb, 5), lambda i: (i, 0)),
            scratch_shapes=[pltpu.VMEM((tb, _P1_PAD), jnp.float32)],
            compiler_params=pltpu.CompilerParams(
                dimension_semantics=("parallel",),
                vmem_limit_bytes=48 * 1024 * 1024),
        )(b1f, b2f, x2, B1, B2, woutT, boutf)
        return logits_pad[:nc]

    logits = run_chunk(x)

    probs = pl.pallas_call(
        _softmax_dim0_kernel,
        out_shape=jax.ShapeDtypeStruct((n, 5), jnp.float32),
        in_specs=[pl.BlockSpec(memory_space=pltpu.MemorySpace.VMEM)],
        out_specs=pl.BlockSpec(memory_space=pltpu.MemorySpace.VMEM),
    )(logits)

    return probs.reshape(n, 1, 5)
